# Initial kernel scaffold; baseline (speedup 1.0000x reference)
import functools

import jax
import jax.numpy as jnp
from jax.experimental import pallas as pl
from jax.experimental.pallas import tpu as pltpu

ALPHA = 0.2
SPAD = 128


def _project_kernel(x_ref, w_ref, a2_ref, hw_ref, s_ref):
    hw = jnp.dot(x_ref[...], w_ref[...], preferred_element_type=jnp.float32)
    hw_ref[...] = hw
    s_ref[...] = jnp.dot(hw, a2_ref[...], preferred_element_type=jnp.float32)


def _attend_kernel(adj_ref, hwj_ref, si_ref, sj_ref, out_ref, rowsum_ref, *,
                   nj):
    j = pl.program_id(2)

    @pl.when(j == 0)
    def _init():
        out_ref[...] = jnp.zeros_like(out_ref)
        rowsum_ref[...] = jnp.zeros_like(rowsum_ref)

    s1 = si_ref[0, :, 0:1]
    s2 = sj_ref[0, :, 1:2]
    logits = s1 + s2.T
    lrelu = jnp.where(logits > 0, logits, ALPHA * logits)
    e = jnp.where(adj_ref[0] > 0, jnp.exp(-lrelu), 0.0)
    rowsum_ref[...] += jnp.sum(e, axis=1, keepdims=True)
    out_ref[0] += jnp.dot(e, hwj_ref[0], preferred_element_type=jnp.float32)

    @pl.when(j == nj - 1)
    def _finalize():
        rs = rowsum_ref[...]
        denom = jnp.where(rs != 0, rs, 1.0)
        hp = out_ref[0] / denom
        out_ref[0] = jnp.where(hp > 0, hp, jnp.expm1(hp))


@jax.jit
def kernel(h, adj, W, a):
    B, N, D = h.shape
    a1 = a[0, :D]
    a2 = a[0, D:]
    a_pair = jnp.zeros((D, SPAD), jnp.float32)
    a_pair = a_pair.at[:, 0].set(a1).at[:, 1].set(a2)

    TP = 512
    hw, S = pl.pallas_call(
        _project_kernel,
        grid=(B * N // TP,),
        in_specs=[
            pl.BlockSpec((TP, D), lambda i: (i, 0)),
            pl.BlockSpec((D, D), lambda i: (0, 0)),
            pl.BlockSpec((D, SPAD), lambda i: (0, 0)),
        ],
        out_specs=[
            pl.BlockSpec((TP, D), lambda i: (i, 0)),
            pl.BlockSpec((TP, SPAD), lambda i: (i, 0)),
        ],
        out_shape=[
            jax.ShapeDtypeStruct((B * N, D), jnp.float32),
            jax.ShapeDtypeStruct((B * N, SPAD), jnp.float32),
        ],
    )(h.reshape(B * N, D), W, a_pair)

    hw3 = hw.reshape(B, N, D)
    S3 = S.reshape(B, N, SPAD)

    TI, TJ = 256, 512
    ni, nj = N // TI, N // TJ
    out = pl.pallas_call(
        functools.partial(_attend_kernel, nj=nj),
        grid=(B, ni, nj),
        in_specs=[
            pl.BlockSpec((1, TI, TJ), lambda b, i, j: (b, i, j)),
            pl.BlockSpec((1, TJ, D), lambda b, i, j: (b, j, 0)),
            pl.BlockSpec((1, TI, SPAD), lambda b, i, j: (b, i, 0)),
            pl.BlockSpec((1, TJ, SPAD), lambda b, i, j: (b, j, 0)),
        ],
        out_specs=pl.BlockSpec((1, TI, D), lambda b, i, j: (b, i, 0)),
        out_shape=jax.ShapeDtypeStruct((B, N, D), jnp.float32),
        scratch_shapes=[pltpu.VMEM((TI, 1), jnp.float32)],
        compiler_params=pltpu.CompilerParams(
            dimension_semantics=("parallel", "parallel", "arbitrary"),
        ),
    )(adj, hw3, S3, S3)
    return out


# speedup vs baseline: 37.1329x; 37.1329x over previous
import functools

import jax
import jax.numpy as jnp
from jax.experimental import pallas as pl
from jax.experimental.pallas import tpu as pltpu

ALPHA = 0.2
SPAD = 128


def _project_kernel(x_ref, w_ref, a2_ref, hw_ref, s_ref):
    hw = jnp.dot(x_ref[...], w_ref[...], preferred_element_type=jnp.float32)
    hw_ref[...] = hw
    s_ref[...] = jnp.dot(hw, a2_ref[...], preferred_element_type=jnp.float32)


def _attend_kernel(adj_ref, hwj_ref, si_ref, sj_ref, out_ref, rowsum_ref, *,
                   nj):
    j = pl.program_id(2)

    @pl.when(j == 0)
    def _init():
        out_ref[...] = jnp.zeros_like(out_ref)
        rowsum_ref[...] = jnp.zeros_like(rowsum_ref)

    s1 = si_ref[0, :, 0:1]
    s2 = sj_ref[0, :, 1:2]
    logits = s1 + s2.T
    lrelu = jnp.where(logits > 0, logits, ALPHA * logits)
    e = jnp.where(adj_ref[0] > 0, jnp.exp(-lrelu), 0.0)
    rowsum_ref[...] += jnp.sum(e, axis=1, keepdims=True)
    out_ref[0] += jnp.dot(e, hwj_ref[0], preferred_element_type=jnp.float32)

    @pl.when(j == nj - 1)
    def _finalize():
        rs = rowsum_ref[...]
        denom = jnp.where(rs != 0, rs, 1.0)
        hp = out_ref[0] / denom
        out_ref[0] = jnp.where(hp > 0, hp, jnp.exp(hp) - 1.0)


@jax.jit
def kernel(h, adj, W, a):
    B, N, D = h.shape
    a1 = a[0, :D]
    a2 = a[0, D:]
    a_pair = jnp.zeros((D, SPAD), jnp.float32)
    a_pair = a_pair.at[:, 0].set(a1).at[:, 1].set(a2)

    TP = 512
    hw, S = pl.pallas_call(
        _project_kernel,
        grid=(B * N // TP,),
        in_specs=[
            pl.BlockSpec((TP, D), lambda i: (i, 0)),
            pl.BlockSpec((D, D), lambda i: (0, 0)),
            pl.BlockSpec((D, SPAD), lambda i: (0, 0)),
        ],
        out_specs=[
            pl.BlockSpec((TP, D), lambda i: (i, 0)),
            pl.BlockSpec((TP, SPAD), lambda i: (i, 0)),
        ],
        out_shape=[
            jax.ShapeDtypeStruct((B * N, D), jnp.float32),
            jax.ShapeDtypeStruct((B * N, SPAD), jnp.float32),
        ],
    )(h.reshape(B * N, D), W, a_pair)

    hw3 = hw.reshape(B, N, D)
    S3 = S.reshape(B, N, SPAD)

    TI, TJ = 256, 512
    ni, nj = N // TI, N // TJ
    out = pl.pallas_call(
        functools.partial(_attend_kernel, nj=nj),
        grid=(B, ni, nj),
        in_specs=[
            pl.BlockSpec((1, TI, TJ), lambda b, i, j: (b, i, j)),
            pl.BlockSpec((1, TJ, D), lambda b, i, j: (b, j, 0)),
            pl.BlockSpec((1, TI, SPAD), lambda b, i, j: (b, i, 0)),
            pl.BlockSpec((1, TJ, SPAD), lambda b, i, j: (b, j, 0)),
        ],
        out_specs=pl.BlockSpec((1, TI, D), lambda b, i, j: (b, i, 0)),
        out_shape=jax.ShapeDtypeStruct((B, N, D), jnp.float32),
        scratch_shapes=[pltpu.VMEM((TI, 1), jnp.float32)],
        compiler_params=pltpu.CompilerParams(
            dimension_semantics=("parallel", "parallel", "arbitrary"),
        ),
    )(adj, hw3, S3, S3)
    return out


# speedup vs baseline: 47.6207x; 1.2824x over previous
import functools

import jax
import jax.numpy as jnp
from jax.experimental import pallas as pl
from jax.experimental.pallas import tpu as pltpu

ALPHA = 0.2
SPAD = 128


def _project_kernel(x_ref, w_ref, a2_ref, hw_ref, s_ref):
    hw = jnp.dot(x_ref[...], w_ref[...], preferred_element_type=jnp.float32)
    hw_ref[...] = hw
    s_ref[...] = jnp.dot(hw, a2_ref[...], preferred_element_type=jnp.float32)


def _attend_kernel(adj_ref, hwj_ref, si_ref, sj_ref, out_ref, rowsum_ref, *,
                   nj):
    j = pl.program_id(2)

    @pl.when(j == 0)
    def _init():
        out_ref[...] = jnp.zeros_like(out_ref)
        rowsum_ref[...] = jnp.zeros_like(rowsum_ref)

    xa = si_ref[0, :, 0:1] + sj_ref[0, :, 1:2].T
    xb = si_ref[0, :, 2:3] + sj_ref[0, :, 3:4].T
    e = jnp.exp2(jnp.minimum(xa, xb)) * adj_ref[0]
    rowsum_ref[...] += jnp.sum(e, axis=1, keepdims=True)
    out_ref[0] += jnp.dot(e, hwj_ref[0], preferred_element_type=jnp.float32)

    @pl.when(j == nj - 1)
    def _finalize():
        rs = rowsum_ref[...]
        denom = jnp.where(rs != 0, rs, 1.0)
        hp = out_ref[0] / denom
        out_ref[0] = jnp.where(hp > 0, hp, jnp.exp(hp) - 1.0)


@jax.jit
def kernel(h, adj, W, a):
    B, N, D = h.shape
    a1 = a[0, :D]
    a2 = a[0, D:]
    neg_log2e = -1.4426950408889634
    a_pair = jnp.zeros((D, SPAD), jnp.float32)
    a_pair = (a_pair
              .at[:, 0].set(neg_log2e * a1)
              .at[:, 1].set(neg_log2e * a2)
              .at[:, 2].set(ALPHA * neg_log2e * a1)
              .at[:, 3].set(ALPHA * neg_log2e * a2))

    TP = 512
    hw, S = pl.pallas_call(
        _project_kernel,
        grid=(B * N // TP,),
        in_specs=[
            pl.BlockSpec((TP, D), lambda i: (i, 0)),
            pl.BlockSpec((D, D), lambda i: (0, 0)),
            pl.BlockSpec((D, SPAD), lambda i: (0, 0)),
        ],
        out_specs=[
            pl.BlockSpec((TP, D), lambda i: (i, 0)),
            pl.BlockSpec((TP, SPAD), lambda i: (i, 0)),
        ],
        out_shape=[
            jax.ShapeDtypeStruct((B * N, D), jnp.float32),
            jax.ShapeDtypeStruct((B * N, SPAD), jnp.float32),
        ],
    )(h.reshape(B * N, D), W, a_pair)

    hw3 = hw.reshape(B, N, D)
    S3 = S.reshape(B, N, SPAD)

    TI, TJ = 256, 1024
    ni, nj = N // TI, N // TJ
    out = pl.pallas_call(
        functools.partial(_attend_kernel, nj=nj),
        grid=(B, ni, nj),
        in_specs=[
            pl.BlockSpec((1, TI, TJ), lambda b, i, j: (b, i, j)),
            pl.BlockSpec((1, TJ, D), lambda b, i, j: (b, j, 0)),
            pl.BlockSpec((1, TI, SPAD), lambda b, i, j: (b, i, 0)),
            pl.BlockSpec((1, TJ, SPAD), lambda b, i, j: (b, j, 0)),
        ],
        out_specs=pl.BlockSpec((1, TI, D), lambda b, i, j: (b, i, 0)),
        out_shape=jax.ShapeDtypeStruct((B, N, D), jnp.float32),
        scratch_shapes=[pltpu.VMEM((TI, 1), jnp.float32)],
        compiler_params=pltpu.CompilerParams(
            dimension_semantics=("parallel", "parallel", "arbitrary"),
        ),
    )(adj, hw3, S3, S3)
    return out


# speedup vs baseline: 57.0736x; 1.1985x over previous
import functools

import jax
import jax.numpy as jnp
from jax.experimental import pallas as pl
from jax.experimental.pallas import tpu as pltpu

ALPHA = 0.2
SPAD = 128


def _project_kernel(x_ref, w_ref, a2_ref, hw16_ref, s_ref):
    hw = jnp.dot(x_ref[...], w_ref[...], preferred_element_type=jnp.float32)
    hw16_ref[...] = hw.astype(jnp.bfloat16)
    s_ref[...] = jnp.dot(hw, a2_ref[...], preferred_element_type=jnp.float32)


def _attend_kernel(adj_ref, hwj_ref, si_ref, sj_ref, out_ref, rowsum_ref, *,
                   nj):
    j = pl.program_id(2)

    @pl.when(j == 0)
    def _init():
        out_ref[...] = jnp.zeros_like(out_ref)
        rowsum_ref[...] = jnp.zeros_like(rowsum_ref)

    x = si_ref[0, :, 0:1] + sj_ref[0, :, 1:2].T
    e = jnp.exp2(jnp.minimum(x, ALPHA * x)) * adj_ref[0]
    rowsum_ref[...] += jnp.sum(e, axis=1, keepdims=True)
    out_ref[0] += jnp.dot(e.astype(jnp.bfloat16), hwj_ref[0],
                          preferred_element_type=jnp.float32)

    @pl.when(j == nj - 1)
    def _finalize():
        rs = rowsum_ref[...]
        denom = jnp.where(rs != 0, rs, 1.0)
        hp = out_ref[0] / denom
        out_ref[0] = jnp.where(hp > 0, hp, jnp.exp(hp) - 1.0)


@jax.jit
def kernel(h, adj, W, a):
    B, N, D = h.shape
    a1 = a[0, :D]
    a2 = a[0, D:]
    neg_log2e = -1.4426950408889634
    a_pair = jnp.zeros((D, SPAD), jnp.float32)
    a_pair = (a_pair
              .at[:, 0].set(neg_log2e * a1)
              .at[:, 1].set(neg_log2e * a2))

    TP = 512
    hw16, S = pl.pallas_call(
        _project_kernel,
        grid=(B * N // TP,),
        in_specs=[
            pl.BlockSpec((TP, D), lambda i: (i, 0)),
            pl.BlockSpec((D, D), lambda i: (0, 0)),
            pl.BlockSpec((D, SPAD), lambda i: (0, 0)),
        ],
        out_specs=[
            pl.BlockSpec((TP, D), lambda i: (i, 0)),
            pl.BlockSpec((TP, SPAD), lambda i: (i, 0)),
        ],
        out_shape=[
            jax.ShapeDtypeStruct((B * N, D), jnp.bfloat16),
            jax.ShapeDtypeStruct((B * N, SPAD), jnp.float32),
        ],
    )(h.reshape(B * N, D), W, a_pair)

    hw3 = hw16.reshape(B, N, D)
    S3 = S.reshape(B, N, SPAD)

    TI, TJ = 256, 1024
    ni, nj = N // TI, N // TJ
    out = pl.pallas_call(
        functools.partial(_attend_kernel, nj=nj),
        grid=(B, ni, nj),
        in_specs=[
            pl.BlockSpec((1, TI, TJ), lambda b, i, j: (b, i, j)),
            pl.BlockSpec((1, TJ, D), lambda b, i, j: (b, j, 0)),
            pl.BlockSpec((1, TI, SPAD), lambda b, i, j: (b, i, 0)),
            pl.BlockSpec((1, TJ, SPAD), lambda b, i, j: (b, j, 0)),
        ],
        out_specs=pl.BlockSpec((1, TI, D), lambda b, i, j: (b, i, 0)),
        out_shape=jax.ShapeDtypeStruct((B, N, D), jnp.float32),
        scratch_shapes=[pltpu.VMEM((TI, 1), jnp.float32)],
        compiler_params=pltpu.CompilerParams(
            dimension_semantics=("parallel", "parallel", "arbitrary"),
        ),
    )(adj, hw3, S3, S3)
    return out


# speedup vs baseline: 83.9211x; 1.4704x over previous
import functools

import jax
import jax.numpy as jnp
from jax.experimental import pallas as pl
from jax.experimental.pallas import tpu as pltpu

ALPHA = 0.2
SPAD = 128


def _project_kernel(x_ref, w_ref, a2_ref, hw16_ref, s_ref):
    hw = jnp.dot(x_ref[...], w_ref[...], preferred_element_type=jnp.float32)
    hw16_ref[...] = hw.astype(jnp.bfloat16)
    s_ref[...] = jnp.dot(hw, a2_ref[...], preferred_element_type=jnp.float32)


def _attend_kernel(adj_ref, hwj_ref, si_ref, sj_ref, out_ref, rowsum_ref, *,
                   nj):
    j = pl.program_id(2)

    @pl.when(j == 0)
    def _init():
        out_ref[...] = jnp.zeros_like(out_ref)
        rowsum_ref[...] = jnp.zeros_like(rowsum_ref)

    x = si_ref[0, :, 0:1] + sj_ref[0, :, 1:2].T
    e = jnp.exp2(jnp.minimum(x, ALPHA * x)) * adj_ref[0]
    rowsum_ref[...] += jnp.sum(e, axis=1, keepdims=True)
    out_ref[0] += jnp.dot(e.astype(jnp.bfloat16), hwj_ref[0],
                          preferred_element_type=jnp.float32)

    @pl.when(j == nj - 1)
    def _finalize():
        rs = rowsum_ref[...]
        denom = jnp.where(rs != 0, rs, 1.0)
        hp = out_ref[0] / denom
        out_ref[0] = jnp.where(hp > 0, hp, jnp.exp(hp) - 1.0)


@jax.jit
def kernel(h, adj, W, a):
    B, N, D = h.shape
    a1 = a[0, :D]
    a2 = a[0, D:]
    neg_log2e = -1.4426950408889634
    a_pair = jnp.zeros((D, SPAD), jnp.float32)
    a_pair = (a_pair
              .at[:, 0].set(neg_log2e * a1)
              .at[:, 1].set(neg_log2e * a2))

    TP = 512
    hw16, S = pl.pallas_call(
        _project_kernel,
        grid=(B * N // TP,),
        in_specs=[
            pl.BlockSpec((TP, D), lambda i: (i, 0)),
            pl.BlockSpec((D, D), lambda i: (0, 0)),
            pl.BlockSpec((D, SPAD), lambda i: (0, 0)),
        ],
        out_specs=[
            pl.BlockSpec((TP, D), lambda i: (i, 0)),
            pl.BlockSpec((TP, SPAD), lambda i: (i, 0)),
        ],
        out_shape=[
            jax.ShapeDtypeStruct((B * N, D), jnp.bfloat16),
            jax.ShapeDtypeStruct((B * N, SPAD), jnp.float32),
        ],
    )(h.reshape(B * N, D), W, a_pair)

    hw3 = hw16.reshape(B, N, D)
    S3 = S.reshape(B, N, SPAD)

    TI, TJ = 512, 1024
    ni, nj = N // TI, N // TJ
    out = pl.pallas_call(
        functools.partial(_attend_kernel, nj=nj),
        grid=(B, ni, nj),
        in_specs=[
            pl.BlockSpec((1, TI, TJ), lambda b, i, j: (b, i, j)),
            pl.BlockSpec((1, TJ, D), lambda b, i, j: (b, j, 0)),
            pl.BlockSpec((1, TI, SPAD), lambda b, i, j: (b, i, 0)),
            pl.BlockSpec((1, TJ, SPAD), lambda b, i, j: (b, j, 0)),
        ],
        out_specs=pl.BlockSpec((1, TI, D), lambda b, i, j: (b, i, 0)),
        out_shape=jax.ShapeDtypeStruct((B, N, D), jnp.float32),
        scratch_shapes=[pltpu.VMEM((TI, 1), jnp.float32)],
        compiler_params=pltpu.CompilerParams(
            dimension_semantics=("parallel", "parallel", "arbitrary"),
        ),
    )(adj, hw3, S3, S3)
    return out


# speedup vs baseline: 103.6542x; 1.2351x over previous
import functools

import jax
import jax.numpy as jnp
from jax.experimental import pallas as pl
from jax.experimental.pallas import tpu as pltpu

ALPHA = 0.2
SPAD = 128


def _project_kernel(x_ref, w_ref, a2_ref, hw16_ref, s_ref):
    hw = jnp.dot(x_ref[...], w_ref[...], preferred_element_type=jnp.float32)
    hw16_ref[...] = hw.astype(jnp.bfloat16)
    s_ref[...] = jnp.dot(hw, a2_ref[...], preferred_element_type=jnp.float32)


def _attend_kernel(adj_ref, hwj_ref, si_ref, sj_ref, out_ref, rowsum_ref, *,
                   nj):
    j = pl.program_id(2)

    @pl.when(j == 0)
    def _init():
        out_ref[...] = jnp.zeros_like(out_ref)
        rowsum_ref[...] = jnp.zeros_like(rowsum_ref)

    x = si_ref[0, :, 0:1] + sj_ref[0, :, 1:2].T
    e = jnp.exp2(jnp.minimum(x, ALPHA * x)) * adj_ref[0]
    rowsum_ref[...] += jnp.sum(e, axis=1, keepdims=True)
    out_ref[0] += jnp.dot(e.astype(jnp.bfloat16), hwj_ref[0],
                          preferred_element_type=jnp.float32)

    @pl.when(j == nj - 1)
    def _finalize():
        rs = rowsum_ref[...]
        denom = jnp.where(rs != 0, rs, 1.0)
        hp = out_ref[0] / denom
        out_ref[0] = jnp.where(hp > 0, hp, jnp.exp(hp) - 1.0)


@jax.jit
def kernel(h, adj, W, a):
    B, N, D = h.shape
    a1 = a[0, :D]
    a2 = a[0, D:]
    neg_log2e = -1.4426950408889634
    a_pair = jnp.zeros((D, SPAD), jnp.float32)
    a_pair = (a_pair
              .at[:, 0].set(neg_log2e * a1)
              .at[:, 1].set(neg_log2e * a2))

    TP = 512
    hw16, S = pl.pallas_call(
        _project_kernel,
        grid=(B * N // TP,),
        in_specs=[
            pl.BlockSpec((TP, D), lambda i: (i, 0)),
            pl.BlockSpec((D, D), lambda i: (0, 0)),
            pl.BlockSpec((D, SPAD), lambda i: (0, 0)),
        ],
        out_specs=[
            pl.BlockSpec((TP, D), lambda i: (i, 0)),
            pl.BlockSpec((TP, SPAD), lambda i: (i, 0)),
        ],
        out_shape=[
            jax.ShapeDtypeStruct((B * N, D), jnp.bfloat16),
            jax.ShapeDtypeStruct((B * N, SPAD), jnp.float32),
        ],
    )(h.reshape(B * N, D), W, a_pair)

    hw3 = hw16.reshape(B, N, D)
    S3 = S.reshape(B, N, SPAD)

    TI, TJ = 512, 2048
    ni, nj = N // TI, N // TJ
    out = pl.pallas_call(
        functools.partial(_attend_kernel, nj=nj),
        grid=(B, ni, nj),
        in_specs=[
            pl.BlockSpec((1, TI, TJ), lambda b, i, j: (b, i, j)),
            pl.BlockSpec((1, TJ, D), lambda b, i, j: (b, j, 0)),
            pl.BlockSpec((1, TI, SPAD), lambda b, i, j: (b, i, 0)),
            pl.BlockSpec((1, TJ, SPAD), lambda b, i, j: (b, j, 0)),
        ],
        out_specs=pl.BlockSpec((1, TI, D), lambda b, i, j: (b, i, 0)),
        out_shape=jax.ShapeDtypeStruct((B, N, D), jnp.float32),
        scratch_shapes=[pltpu.VMEM((TI, 1), jnp.float32)],
        compiler_params=pltpu.CompilerParams(
            dimension_semantics=("parallel", "parallel", "arbitrary"),
        ),
    )(adj, hw3, S3, S3)
    return out


# speedup vs baseline: 120.7043x; 1.1645x over previous
import functools

import jax
import jax.numpy as jnp
from jax.experimental import pallas as pl
from jax.experimental.pallas import tpu as pltpu

ALPHA = 0.2
SPAD = 128
DPAD = 384


def _project_kernel(x_ref, w_ref, a2_ref, hw16_ref, s_ref):
    hw = jnp.dot(x_ref[...], w_ref[...], preferred_element_type=jnp.float32)
    tp = hw.shape[0]
    ones = jnp.ones((tp, 1), jnp.float32)
    zeros = jnp.zeros((tp, DPAD - hw.shape[1] - 1), jnp.float32)
    hw16_ref[...] = jnp.concatenate([hw, ones, zeros], axis=1).astype(jnp.bfloat16)
    s_ref[...] = jnp.dot(hw, a2_ref[...], preferred_element_type=jnp.float32)


def _attend_kernel(adj_ref, hwj_ref, si_ref, sj_ref, out_ref, acc_ref, *,
                   nj, d):
    j = pl.program_id(2)

    x = si_ref[0, :, 0:1] + sj_ref[0, :, 1:2].T
    e = jnp.exp2(jnp.minimum(x, ALPHA * x)) * adj_ref[0]
    prod = jnp.dot(e.astype(jnp.bfloat16), hwj_ref[0],
                   preferred_element_type=jnp.float32)
    if nj > 1:
        @pl.when(j == 0)
        def _init():
            acc_ref[...] = prod

        @pl.when(j > 0)
        def _accum():
            acc_ref[...] += prod

    @pl.when(j == nj - 1)
    def _finalize():
        acc = acc_ref[...] if nj > 1 else prod
        rs = acc[:, d:d + 1]
        denom = jnp.where(rs != 0, rs, 1.0)
        hp = acc[:, :d] / denom
        out_ref[0] = jnp.where(hp > 0, hp, jnp.exp(hp) - 1.0)


@jax.jit
def kernel(h, adj, W, a):
    B, N, D = h.shape
    a1 = a[0, :D]
    a2 = a[0, D:]
    neg_log2e = -1.4426950408889634
    a_pair = jnp.zeros((D, SPAD), jnp.float32)
    a_pair = (a_pair
              .at[:, 0].set(neg_log2e * a1)
              .at[:, 1].set(neg_log2e * a2))

    TP = 512
    hw16, S = pl.pallas_call(
        _project_kernel,
        grid=(B * N // TP,),
        in_specs=[
            pl.BlockSpec((TP, D), lambda i: (i, 0)),
            pl.BlockSpec((D, D), lambda i: (0, 0)),
            pl.BlockSpec((D, SPAD), lambda i: (0, 0)),
        ],
        out_specs=[
            pl.BlockSpec((TP, DPAD), lambda i: (i, 0)),
            pl.BlockSpec((TP, SPAD), lambda i: (i, 0)),
        ],
        out_shape=[
            jax.ShapeDtypeStruct((B * N, DPAD), jnp.bfloat16),
            jax.ShapeDtypeStruct((B * N, SPAD), jnp.float32),
        ],
    )(h.reshape(B * N, D), W, a_pair)

    hw3 = hw16.reshape(B, N, DPAD)
    S3 = S.reshape(B, N, SPAD)

    TI, TJ = 1024, 2048
    ni, nj = N // TI, N // TJ
    out = pl.pallas_call(
        functools.partial(_attend_kernel, nj=nj, d=D),
        grid=(B, ni, nj),
        in_specs=[
            pl.BlockSpec((1, TI, TJ), lambda b, i, j: (b, i, j)),
            pl.BlockSpec((1, TJ, DPAD), lambda b, i, j: (b, j, 0)),
            pl.BlockSpec((1, TI, SPAD), lambda b, i, j: (b, i, 0)),
            pl.BlockSpec((1, TJ, SPAD), lambda b, i, j: (b, j, 0)),
        ],
        out_specs=pl.BlockSpec((1, TI, D), lambda b, i, j: (b, i, 0)),
        out_shape=jax.ShapeDtypeStruct((B, N, D), jnp.float32),
        scratch_shapes=[pltpu.VMEM((TI, DPAD), jnp.float32)],
        compiler_params=pltpu.CompilerParams(
            dimension_semantics=("parallel", "parallel", "arbitrary"),
        ),
    )(adj, hw3, S3, S3)
    return out


# speedup vs baseline: 137.3361x; 1.1378x over previous
import functools

import jax
import jax.numpy as jnp
from jax.experimental import pallas as pl
from jax.experimental.pallas import tpu as pltpu

ALPHA = 0.2
SPAD = 8


def _project_kernel(x_ref, w_ref, a2_ref, hw16_ref, s_ref):
    hw = jnp.dot(x_ref[...], w_ref[...], preferred_element_type=jnp.float32)
    hw16_ref[...] = hw.astype(jnp.bfloat16)
    s_ref[...] = jnp.dot(hw, a2_ref[...], preferred_element_type=jnp.float32)


def _attend_kernel(adj_ref, hwj_ref, s_ref, out_ref, *, ti):
    i = pl.program_id(1)
    si = s_ref[0, pl.ds(i * ti, ti), 0:1]
    sj = s_ref[0, :, 1:2]
    x = si + sj.T
    e = jnp.exp2(jnp.minimum(x, ALPHA * x)) * adj_ref[0]
    rowsum = jnp.sum(e, axis=1, keepdims=True)
    denom = jnp.where(rowsum != 0, rowsum, 1.0)
    prod = jnp.dot(e.astype(jnp.bfloat16), hwj_ref[0],
                   preferred_element_type=jnp.float32)
    hp = prod / denom
    out_ref[0] = jnp.where(hp > 0, hp, jnp.exp(hp) - 1.0)


@jax.jit
def kernel(h, adj, W, a):
    B, N, D = h.shape
    a1 = a[0, :D]
    a2 = a[0, D:]
    neg_log2e = -1.4426950408889634
    a_pair = jnp.zeros((D, SPAD), jnp.float32)
    a_pair = (a_pair
              .at[:, 0].set(neg_log2e * a1)
              .at[:, 1].set(neg_log2e * a2))

    TP = 512
    hw16, S = pl.pallas_call(
        _project_kernel,
        grid=(B * N // TP,),
        in_specs=[
            pl.BlockSpec((TP, D), lambda i: (i, 0)),
            pl.BlockSpec((D, D), lambda i: (0, 0)),
            pl.BlockSpec((D, SPAD), lambda i: (0, 0)),
        ],
        out_specs=[
            pl.BlockSpec((TP, D), lambda i: (i, 0)),
            pl.BlockSpec((TP, SPAD), lambda i: (i, 0)),
        ],
        out_shape=[
            jax.ShapeDtypeStruct((B * N, D), jnp.bfloat16),
            jax.ShapeDtypeStruct((B * N, SPAD), jnp.float32),
        ],
    )(h.reshape(B * N, D), W, a_pair)

    hw3 = hw16.reshape(B, N, D)
    S3 = S.reshape(B, N, SPAD)

    TI = 512
    ni = N // TI
    out = pl.pallas_call(
        functools.partial(_attend_kernel, ti=TI),
        grid=(B, ni),
        in_specs=[
            pl.BlockSpec((1, TI, N), lambda b, i: (b, i, 0)),
            pl.BlockSpec((1, N, D), lambda b, i: (b, 0, 0)),
            pl.BlockSpec((1, N, SPAD), lambda b, i: (b, 0, 0)),
        ],
        out_specs=pl.BlockSpec((1, TI, D), lambda b, i: (b, i, 0)),
        out_shape=jax.ShapeDtypeStruct((B, N, D), jnp.float32),
        compiler_params=pltpu.CompilerParams(
            dimension_semantics=("parallel", "arbitrary"),
        ),
    )(adj, hw3, S3)
    return out
